# baseline (speedup 1.0000x reference)
import functools

import jax
import jax.numpy as jnp
from jax.experimental import pallas as pl

B, N, M = 4, 2048, 2048
K, K1 = 10, 64
SLBP_ITER, COST_SCALE, ALPHA = 5, 50.0, 0.1
EMB, HEADS, FF = 64, 4, 1024
DH = EMB // HEADS
R = 256
NBLK = N // R
BIG = 1e30
_INTERPRET = False


def _ln(x):
    mu = jnp.mean(x, axis=-1, keepdims=True)
    xc = x - mu
    var = jnp.mean(xc * xc, axis=-1, keepdims=True)
    return xc / jnp.sqrt(var + 1e-5)


def _knn_body(x_ref, xt_ref, idx_ref):
    rb = pl.program_id(1)
    base = rb * R
    xb = x_ref[0]
    cols = jax.lax.broadcasted_iota(jnp.int32, (R, N), 1).astype(jnp.float32)
    d = jnp.zeros((R, N), jnp.float32)
    for c in range(3):
        diff = xb[:, c:c + 1] - xt_ref[0, c:c + 1, :]
        d = d + diff * diff
    rows = base.astype(jnp.float32) + jax.lax.broadcasted_iota(
        jnp.int32, (R, N), 0).astype(jnp.float32)
    d = jnp.where(cols == rows, BIG, d)
    kcols = jax.lax.broadcasted_iota(jnp.int32, (1, K), 1).astype(jnp.float32)

    def step(c, carry):
        d, acc = carry
        m = jnp.min(d, axis=1, keepdims=True)
        sel = jnp.where(d <= m, cols, jnp.float32(N))
        am = jnp.min(sel, axis=1, keepdims=True)
        d = jnp.where(cols == am, BIG, d)
        acc = acc + jnp.where(kcols == c.astype(jnp.float32), am, 0.0)
        return d, acc

    _, acc = jax.lax.fori_loop(0, K, step,
                               (d, jnp.zeros((R, K), jnp.float32)))
    idx_ref[0] = acc.astype(jnp.int32)


def _knn(x, xt):
    return pl.pallas_call(
        _knn_body,
        grid=(B, NBLK),
        in_specs=[pl.BlockSpec((1, R, 3), lambda b, r: (b, r, 0)),
                  pl.BlockSpec((1, 3, N), lambda b, r: (b, 0, 0))],
        out_specs=pl.BlockSpec((1, R, K), lambda b, r: (b, r, 0)),
        out_shape=jax.ShapeDtypeStruct((B, N, K), jnp.int32),
        interpret=_INTERPRET,
    )(x, xt)


def _edge_body(x_ref, idx_ref, w_ref, b_ref, out_ref):
    rb = pl.program_id(1)
    base = rb * R
    w = w_ref[...]
    g_full = jnp.dot(x_ref[0], w[3:6],
                     preferred_element_type=jnp.float32, precision=jax.lax.Precision.HIGHEST)
    xb = x_ref[0, pl.ds(base, R), :]
    a = jnp.dot(xb, w[0:3] - w[3:6],
                preferred_element_type=jnp.float32, precision=jax.lax.Precision.HIGHEST) + b_ref[...]
    cols = jax.lax.broadcasted_iota(jnp.int32, (R, N), 1).astype(jnp.float32)
    acc = jnp.zeros((R, EMB), jnp.float32)
    for j in range(K):
        idxj = idx_ref[0, :, j:j + 1].astype(jnp.float32)
        oh = jnp.where(cols == idxj, 1.0, 0.0)
        gj = jnp.dot(oh, g_full, preferred_element_type=jnp.float32, precision=jax.lax.Precision.HIGHEST)
        acc = jnp.maximum(acc, jnp.maximum(a + gj, 0.0))
    out_ref[0] = acc


def _edge(x, idx, w, b2):
    return pl.pallas_call(
        _edge_body,
        grid=(B, NBLK),
        in_specs=[pl.BlockSpec((1, N, 3), lambda b, r: (b, 0, 0)),
                  pl.BlockSpec((1, R, K), lambda b, r: (b, r, 0)),
                  pl.BlockSpec((6, EMB), lambda b, r: (0, 0)),
                  pl.BlockSpec((1, EMB), lambda b, r: (0, 0))],
        out_specs=pl.BlockSpec((1, R, EMB), lambda b, r: (b, r, 0)),
        out_shape=jax.ShapeDtypeStruct((B, N, EMB), jnp.float32),
        interpret=_INTERPRET,
    )(x, idx, w, b2)


def _cross_body(qf_ref, kvf_ref, wq_ref, wk_ref, wv_ref, wo_ref,
                w1_ref, b1_ref, w2_ref, b2_ref, out_ref):
    kv = kvf_ref[0]
    kmat = jnp.dot(kv, wk_ref[...], preferred_element_type=jnp.float32, precision=jax.lax.Precision.HIGHEST)
    vmat = jnp.dot(kv, wv_ref[...], preferred_element_type=jnp.float32, precision=jax.lax.Precision.HIGHEST)
    scale = 1.0 / (DH ** 0.5)
    for cidx in range(NBLK):
        qfb = qf_ref[0, cidx * R:(cidx + 1) * R, :]
        q = jnp.dot(qfb, wq_ref[...], preferred_element_type=jnp.float32, precision=jax.lax.Precision.HIGHEST)
        outs = []
        for h in range(HEADS):
            qh = q[:, h * DH:(h + 1) * DH]
            kh = kmat[:, h * DH:(h + 1) * DH]
            vh = vmat[:, h * DH:(h + 1) * DH]
            lg = jax.lax.dot_general(
                qh, kh, (((1,), (1,)), ((), ())),
                preferred_element_type=jnp.float32, precision=jax.lax.Precision.HIGHEST) * scale
            mx = jnp.max(lg, axis=1, keepdims=True)
            e = jnp.exp(lg - mx)
            att = e / jnp.sum(e, axis=1, keepdims=True)
            outs.append(jnp.dot(att, vh, preferred_element_type=jnp.float32, precision=jax.lax.Precision.HIGHEST))
        o = jnp.concatenate(outs, axis=1)
        h1 = _ln(qfb + jnp.dot(o, wo_ref[...],
                               preferred_element_type=jnp.float32, precision=jax.lax.Precision.HIGHEST))
        ff = jnp.dot(
            jnp.maximum(jnp.dot(h1, w1_ref[...],
                                preferred_element_type=jnp.float32, precision=jax.lax.Precision.HIGHEST)
                        + b1_ref[...], 0.0),
            w2_ref[...], preferred_element_type=jnp.float32, precision=jax.lax.Precision.HIGHEST) + b2_ref[...]
        out_ref[0, cidx * R:(cidx + 1) * R, :] = _ln(h1 + ff)


def _cross(qf, kvf, wq, wk, wv, wo, w1, b1, w2, b2):
    return pl.pallas_call(
        _cross_body,
        grid=(B,),
        in_specs=[pl.BlockSpec((1, N, EMB), lambda b: (b, 0, 0)),
                  pl.BlockSpec((1, M, EMB), lambda b: (b, 0, 0)),
                  pl.BlockSpec((EMB, EMB), lambda b: (0, 0)),
                  pl.BlockSpec((EMB, EMB), lambda b: (0, 0)),
                  pl.BlockSpec((EMB, EMB), lambda b: (0, 0)),
                  pl.BlockSpec((EMB, EMB), lambda b: (0, 0)),
                  pl.BlockSpec((EMB, FF), lambda b: (0, 0)),
                  pl.BlockSpec((1, FF), lambda b: (0, 0)),
                  pl.BlockSpec((FF, EMB), lambda b: (0, 0)),
                  pl.BlockSpec((1, EMB), lambda b: (0, 0))],
        out_specs=pl.BlockSpec((1, N, EMB), lambda b: (b, 0, 0)),
        out_shape=jax.ShapeDtypeStruct((B, N, EMB), jnp.float32),
        interpret=_INTERPRET,
    )(qf, kvf, wq, wk, wv, wo, w1, b1, w2, b2)


def _cand_body(src_ref, tgt_ref, tgtT_ref, es_ref, etT_ref,
               disp_ref, dc_ref):
    srcb = src_ref[0]
    cols = jax.lax.broadcasted_iota(jnp.int32, (R, M), 1).astype(jnp.float32)
    d = jnp.zeros((R, M), jnp.float32)
    for c in range(3):
        diff = srcb[:, c:c + 1] - tgtT_ref[0, c:c + 1, :]
        d = d + diff * diff
    esb = es_ref[0]
    etT = etT_ref[0]
    dot = jnp.dot(esb, etT, preferred_element_type=jnp.float32, precision=jax.lax.Precision.HIGHEST)
    esq = jnp.sum(esb * esb, axis=1, keepdims=True)
    tsq = jnp.sum(etT * etT, axis=0, keepdims=True)
    dcost = (COST_SCALE / EMB) * (esq + (tsq - 2.0 * dot))
    k1cols = jax.lax.broadcasted_iota(jnp.int32, (1, K1), 1).astype(jnp.float32)
    tgt = tgt_ref[0]

    def step(c, carry):
        d, dx, dy, dz, dca = carry
        m = jnp.min(d, axis=1, keepdims=True)
        sel = jnp.where(d <= m, cols, jnp.float32(M))
        am = jnp.min(sel, axis=1, keepdims=True)
        oh = jnp.where(cols == am, 1.0, 0.0)
        d = jnp.where(cols == am, BIG, d)
        trow = jnp.dot(oh, tgt, preferred_element_type=jnp.float32, precision=jax.lax.Precision.HIGHEST)
        dval = jnp.sum(oh * dcost, axis=1, keepdims=True)
        msk = k1cols == c.astype(jnp.float32)
        dsp = trow - srcb
        dx = dx + jnp.where(msk, dsp[:, 0:1], 0.0)
        dy = dy + jnp.where(msk, dsp[:, 1:2], 0.0)
        dz = dz + jnp.where(msk, dsp[:, 2:3], 0.0)
        dca = dca + jnp.where(msk, dval, 0.0)
        return d, dx, dy, dz, dca

    z = jnp.zeros((R, K1), jnp.float32)
    _, dx, dy, dz, dca = jax.lax.fori_loop(0, K1, step, (d, z, z, z, z))
    disp_ref[0, :, 0:K1] = dx
    disp_ref[0, :, K1:2 * K1] = dy
    disp_ref[0, :, 2 * K1:3 * K1] = dz
    dc_ref[0] = dca


def _cand(src, tgt, tgtT, es, etT):
    return pl.pallas_call(
        _cand_body,
        grid=(B, NBLK),
        in_specs=[pl.BlockSpec((1, R, 3), lambda b, r: (b, r, 0)),
                  pl.BlockSpec((1, M, 3), lambda b, r: (b, 0, 0)),
                  pl.BlockSpec((1, 3, M), lambda b, r: (b, 0, 0)),
                  pl.BlockSpec((1, R, EMB), lambda b, r: (b, r, 0)),
                  pl.BlockSpec((1, EMB, M), lambda b, r: (b, 0, 0))],
        out_specs=[pl.BlockSpec((1, R, 3 * K1), lambda b, r: (b, r, 0)),
                   pl.BlockSpec((1, R, K1), lambda b, r: (b, r, 0))],
        out_shape=[jax.ShapeDtypeStruct((B, N, 3 * K1), jnp.float32),
                   jax.ShapeDtypeStruct((B, N, K1), jnp.float32)],
        interpret=_INTERPRET,
    )(src, tgt, tgtT, es, etT)


def _nbprep_body(disp_ref, nidx_ref, nbd_ref, nbs_ref):
    cols = jax.lax.broadcasted_iota(jnp.int32, (R, N), 1).astype(jnp.float32)
    dispf = disp_ref[0]
    for j in range(K):
        idxj = nidx_ref[0, :, j:j + 1].astype(jnp.float32)
        oh = jnp.where(cols == idxj, 1.0, 0.0)
        nbd = jnp.dot(oh, dispf, preferred_element_type=jnp.float32, precision=jax.lax.Precision.HIGHEST)
        nbd_ref[0, j] = nbd
        nbs_ref[0, j] = ALPHA * (nbd[:, 0:K1] ** 2
                                 + nbd[:, K1:2 * K1] ** 2
                                 + nbd[:, 2 * K1:3 * K1] ** 2)


def _nbprep(disp, nidx):
    return pl.pallas_call(
        _nbprep_body,
        grid=(B, NBLK),
        in_specs=[pl.BlockSpec((1, N, 3 * K1), lambda b, r: (b, 0, 0)),
                  pl.BlockSpec((1, R, K), lambda b, r: (b, r, 0))],
        out_specs=[pl.BlockSpec((1, K, R, 3 * K1),
                                lambda b, r: (b, 0, r, 0)),
                   pl.BlockSpec((1, K, R, K1), lambda b, r: (b, 0, r, 0))],
        out_shape=[jax.ShapeDtypeStruct((B, K, N, 3 * K1), jnp.float32),
                   jax.ShapeDtypeStruct((B, K, N, K1), jnp.float32)],
        interpret=_INTERPRET,
    )(disp, nidx)


def _lbp_body(bel_ref, dc_ref, disp_ref, nidx_ref, nbd_ref, nbs_ref,
              out_ref):
    cols = jax.lax.broadcasted_iota(jnp.int32, (R, N), 1).astype(jnp.float32)
    bel = bel_ref[0]
    dispd = [disp_ref[0, :, dd * K1:(dd + 1) * K1] for dd in range(3)]
    dn = ALPHA * (dispd[0] ** 2 + dispd[1] ** 2 + dispd[2] ** 2)
    nidxf = nidx_ref[0].astype(jnp.float32)
    jcols = jax.lax.broadcasted_iota(jnp.int32, (1, K), 1).astype(jnp.float32)

    def jstep(j, msum):
        jf = j.astype(jnp.float32)
        idxj = jnp.sum(jnp.where(jcols == jf, nidxf, 0.0),
                       axis=1, keepdims=True)
        oh = jnp.where(cols == idxj, 1.0, 0.0)
        nb_b = jnp.dot(oh, bel, preferred_element_type=jnp.float32, precision=jax.lax.Precision.HIGHEST)
        ca = nb_b + nbs_ref[0, j]
        nbdj = nbd_ref[0, j]
        cross = jnp.zeros((R, K1, K1), jnp.float32)
        for dd in range(3):
            nbd_d = nbdj[:, dd * K1:(dd + 1) * K1]
            cross = cross + nbd_d[:, :, None] * dispd[dd][:, None, :]
        t3 = ca[:, :, None] + (-2.0 * ALPHA) * cross
        return msum + jnp.min(t3, axis=1) + dn

    msum = jax.lax.fori_loop(0, K, jstep, jnp.zeros((R, K1), jnp.float32))
    out_ref[0] = dc_ref[0] + msum


def _lbp(bel, dc, disp, nidx, nbd, nbs):
    return pl.pallas_call(
        _lbp_body,
        grid=(B, NBLK),
        in_specs=[pl.BlockSpec((1, N, K1), lambda b, r: (b, 0, 0)),
                  pl.BlockSpec((1, R, K1), lambda b, r: (b, r, 0)),
                  pl.BlockSpec((1, R, 3 * K1), lambda b, r: (b, r, 0)),
                  pl.BlockSpec((1, R, K), lambda b, r: (b, r, 0)),
                  pl.BlockSpec((1, K, R, 3 * K1),
                               lambda b, r: (b, 0, r, 0)),
                  pl.BlockSpec((1, K, R, K1), lambda b, r: (b, 0, r, 0))],
        out_specs=pl.BlockSpec((1, R, K1), lambda b, r: (b, r, 0)),
        out_shape=jax.ShapeDtypeStruct((B, N, K1), jnp.float32),
        interpret=_INTERPRET,
    )(bel, dc, disp, nidx, nbd, nbs)


def _final_body(bel_ref, disp_ref, src_ref, out_ref):
    nb = -bel_ref[0]
    mx = jnp.max(nb, axis=1, keepdims=True)
    e = jnp.exp(nb - mx)
    w = e / jnp.sum(e, axis=1, keepdims=True)
    for dd in range(3):
        od = jnp.sum(w * disp_ref[0, :, dd * K1:(dd + 1) * K1],
                     axis=1, keepdims=True)
        out_ref[0, :, dd:dd + 1] = src_ref[0, :, dd:dd + 1] + od


def _final(bel, disp, src):
    return pl.pallas_call(
        _final_body,
        grid=(B,),
        in_specs=[pl.BlockSpec((1, N, K1), lambda b: (b, 0, 0)),
                  pl.BlockSpec((1, N, 3 * K1), lambda b: (b, 0, 0)),
                  pl.BlockSpec((1, N, 3), lambda b: (b, 0, 0))],
        out_specs=pl.BlockSpec((1, N, 3), lambda b: (b, 0, 0)),
        out_shape=jax.ShapeDtypeStruct((B, N, 3), jnp.float32),
        interpret=_INTERPRET,
    )(bel, disp, src)


def kernel(source, target, W_edge, b_edge, Wq, Wk, Wv, Wo,
           W_ff1, b_ff1, W_ff2, b_ff2):
    srcT = jnp.transpose(source, (0, 2, 1))
    tgtT = jnp.transpose(target, (0, 2, 1))
    be2 = b_edge.reshape(1, EMB)
    b1 = b_ff1.reshape(1, FF)
    b2 = b_ff2.reshape(1, EMB)
    nidx_s = _knn(source, srcT)
    nidx_t = _knn(target, tgtT)
    fs = _edge(source, nidx_s, W_edge, be2)
    ft = _edge(target, nidx_t, W_edge, be2)
    es = _cross(fs, ft, Wq, Wk, Wv, Wo, W_ff1, b1, W_ff2, b2)
    et = _cross(ft, fs, Wq, Wk, Wv, Wo, W_ff1, b1, W_ff2, b2)
    etT = jnp.transpose(et, (0, 2, 1))
    disp, dc = _cand(source, target, tgtT, es, etT)
    nbd, nbs = _nbprep(disp, nidx_s)
    bel = dc
    for _ in range(SLBP_ITER):
        bel = _lbp(bel, dc, disp, nidx_s, nbd, nbs)
    return _final(bel, disp, source)
